# baseline (speedup 1.0000x reference)
import functools

import jax
import jax.numpy as jnp
from jax import lax
from jax.experimental import pallas as pl
from jax.experimental.pallas import tpu as pltpu
from jax.experimental.pallas import tpu_sc as plsc

N = 10000
M = 12
D = 64
DN = 41
NPAD = 10240
BPAD = M * NPAD
NW = 32
CPW = 30
CHUNK = 128
RB = 1000
NBC = N // RB
RB2 = 2000
CG = 4
NCRY = 200
NA = 50
EPS = 1e-5


def _softplus(v):
    return jnp.maximum(v, 0.0) + jnp.log(1.0 + jnp.exp(-jnp.abs(v)))


def _sigmoid(v):
    t = jnp.exp(-jnp.abs(v))
    return jnp.where(v >= 0, 1.0 / (1.0 + t), t / (1.0 + t))


def _dotT(a, b):
    return lax.dot_general(a, b, (((1,), (1,)), ((), ())),
                           preferred_element_type=jnp.float32)


def _sc_gather(bv, idx3):
    mesh = plsc.VectorSubcoreMesh(core_axis_name="c", subcore_axis_name="s")

    @functools.partial(
        pl.kernel,
        mesh=mesh,
        out_type=jax.ShapeDtypeStruct((BPAD, 2 * D), jnp.float32),
        scratch_types=[
            pltpu.VMEM((CPW, CHUNK), jnp.int32),
            pltpu.VMEM((CHUNK, 2 * D), jnp.float32),
            pltpu.VMEM((CHUNK, 2 * D), jnp.float32),
            pltpu.VMEM_SHARED((N, 2 * D), jnp.float32),
            pltpu.SemaphoreType.DMA,
            pltpu.SemaphoreType.DMA,
            pltpu.SemaphoreType.DMA,
            pltpu.SemaphoreType.DMA,
        ],
    )
    def k(x_hbm, idx_hbm, out_hbm, idx_v, buf0, buf1, tab, gi0, gi1, go0, go1):
        sid = lax.axis_index("s")
        wid = sid * 2 + lax.axis_index("c")
        base = wid * (CPW * CHUNK)

        @pl.when(sid < 10)
        def _():
            pltpu.sync_copy(x_hbm.at[pl.ds(sid * 1000, 1000)],
                            tab.at[pl.ds(sid * 1000, 1000)])

        pltpu.sync_copy(idx_hbm.at[wid], idx_v)
        plsc.subcore_barrier()

        def g_start(c, buf, sem):
            pltpu.async_copy(tab.at[idx_v.at[c]], buf, sem)

        def g_wait(c, buf, sem):
            pltpu.make_async_copy(tab.at[idx_v.at[c]], buf, sem).wait()

        def s_start(c, buf, sem):
            pltpu.async_copy(buf, out_hbm.at[pl.ds(base + c * CHUNK, CHUNK)], sem)

        def s_wait(buf, sem):
            pltpu.make_async_copy(buf, out_hbm.at[pl.ds(base, CHUNK)], sem).wait()

        g_start(0, buf0, gi0)

        def outer(o, carry):
            c0 = 2 * o
            @pl.when(o > 0)
            def _():
                s_wait(buf1, go1)

            g_start(c0 + 1, buf1, gi1)
            g_wait(c0, buf0, gi0)
            s_start(c0, buf0, go0)

            @pl.when(o < CPW // 2 - 1)
            def _():
                s_wait(buf0, go0)
                g_start(c0 + 2, buf0, gi0)

            g_wait(c0 + 1, buf1, gi1)
            s_start(c0 + 1, buf1, go1)
            return carry

        lax.fori_loop(0, CPW // 2, outer, 0)
        s_wait(buf0, go0)
        s_wait(buf1, go1)

    return k(bv, idx3)


def _embed(atom_fea, W_emb):
    def body(a_ref, w_ref, o_ref):
        o_ref[...] = _dotT(a_ref[...], w_ref[...])

    return pl.pallas_call(
        body,
        grid=(N // RB2,),
        in_specs=[
            pl.BlockSpec((RB2, 92), lambda j: (j, 0)),
            pl.BlockSpec((D, 92), lambda j: (0, 0)),
        ],
        out_specs=pl.BlockSpec((RB2, D), lambda j: (j, 0)),
        out_shape=jax.ShapeDtypeStruct((N, D), jnp.float32),
    )(atom_fea, W_emb)


def _neighbor_transform(x, Wn):
    def body(x_ref, w_ref, o_ref):
        o_ref[...] = _dotT(x_ref[...], w_ref[...])

    return pl.pallas_call(
        body,
        grid=(N // RB2,),
        in_specs=[
            pl.BlockSpec((RB2, D), lambda j: (j, 0)),
            pl.BlockSpec((2 * D, D), lambda j: (0, 0)),
        ],
        out_specs=pl.BlockSpec((RB2, 2 * D), lambda j: (j, 0)),
        out_shape=jax.ShapeDtypeStruct((N, 2 * D), jnp.float32),
    )(x, Wn)


def _conv_layer(x, g0, nbr2, Ws, We, bful, g1, b1):
    def body(x_ref, g0_ref, nf_ref, ws_ref, we_ref, b_ref,
             g1_ref, b1_ref, ns_ref, st_ref, acc):
        ph = pl.program_id(0)
        j = pl.program_id(1)

        @pl.when((ph == 0) & (j == 0))
        def _():
            acc[...] = jnp.zeros_like(acc)

        @pl.when((ph == 1) & (j == 0))
        def _():
            cnt = float(N * M)
            mean = acc[0:1, :] / cnt
            var = acc[1:2, :] / cnt - mean * mean
            acc[2:3, :] = mean
            acc[3:4, :] = lax.rsqrt(var + EPS)

        a = _dotT(x_ref[...], ws_ref[...]) + b_ref[...]

        nf = nf_ref[...]
        gs = []
        for m in range(M):
            g = a + g0_ref[m] + _dotT(nf[:, m * DN:(m + 1) * DN], we_ref[...])
            gs.append(g)

        @pl.when(ph == 0)
        def _():
            s = gs[0]
            s2 = gs[0] * gs[0]
            for m in range(1, M):
                s = s + gs[m]
                s2 = s2 + gs[m] * gs[m]
            acc[0:1, :] += jnp.sum(s, axis=0, keepdims=True)
            acc[1:2, :] += jnp.sum(s2, axis=0, keepdims=True)

        @pl.when(ph == 1)
        def _():
            mean = acc[2:3, :]
            scale = acc[3:4, :] * g1_ref[...]
            shift = b1_ref[...]
            ns = jnp.zeros((RB, D), jnp.float32)
            for m in range(M):
                ghat = (gs[m] - mean) * scale + shift
                ns = ns + _sigmoid(ghat[:, :D]) * _softplus(ghat[:, D:])
            ns_ref[...] = ns
            acc[4:5, :D] += jnp.sum(ns, axis=0, keepdims=True)
            acc[5:6, :D] += jnp.sum(ns * ns, axis=0, keepdims=True)

        @pl.when((ph == 1) & (j == NBC - 1))
        def _():
            st_ref[...] = acc[...]

    return pl.pallas_call(
        body,
        grid=(2, NBC),
        in_specs=[
            pl.BlockSpec((RB, D), lambda p, j: (j, 0)),
            pl.BlockSpec((M, RB, 2 * D), lambda p, j: (0, j, 0)),
            pl.BlockSpec((RB, M * DN), lambda p, j: (j, 0)),
            pl.BlockSpec((2 * D, D), lambda p, j: (0, 0)),
            pl.BlockSpec((2 * D, DN), lambda p, j: (0, 0)),
            pl.BlockSpec((1, 2 * D), lambda p, j: (0, 0)),
            pl.BlockSpec((1, 2 * D), lambda p, j: (0, 0)),
            pl.BlockSpec((1, 2 * D), lambda p, j: (0, 0)),
        ],
        out_specs=[
            pl.BlockSpec((RB, D), lambda p, j: (j, 0)),
            pl.BlockSpec((8, 2 * D), lambda p, j: (0, 0)),
        ],
        out_shape=[
            jax.ShapeDtypeStruct((N, D), jnp.float32),
            jax.ShapeDtypeStruct((8, 2 * D), jnp.float32),
        ],
        scratch_shapes=[pltpu.VMEM((8, 2 * D), jnp.float32)],
    )(x, g0, nbr2, Ws, We, bful, g1, b1)


def _bn2_apply(x, ns, stats, g2, b2):
    def body(x_ref, ns_ref, st_ref, g2_ref, b2_ref, o_ref):
        m2 = st_ref[4:5, :D] / float(N)
        v2 = st_ref[5:6, :D] / float(N) - m2 * m2
        sc = lax.rsqrt(v2 + EPS) * g2_ref[...]
        o_ref[...] = _softplus(x_ref[...] + (ns_ref[...] - m2) * sc + b2_ref[...])

    return pl.pallas_call(
        body,
        grid=(N // RB2,),
        in_specs=[
            pl.BlockSpec((RB2, D), lambda j: (j, 0)),
            pl.BlockSpec((RB2, D), lambda j: (j, 0)),
            pl.BlockSpec((8, 2 * D), lambda j: (0, 0)),
            pl.BlockSpec((1, D), lambda j: (0, 0)),
            pl.BlockSpec((1, D), lambda j: (0, 0)),
        ],
        out_specs=pl.BlockSpec((RB2, D), lambda j: (j, 0)),
        out_shape=jax.ShapeDtypeStruct((N, D), jnp.float32),
    )(x, ns, stats, g2, b2)


def _decoder(bt3, fc_adj_W, fc_edge_W, fc_atom_W, fc_atom_b,
             fc1_W, fc1_b, fc_adj_b, fc2_W, fc2_b, fc_edge_b):
    def body(bt_ref, adj_ref, edg_ref, aw_ref, ab_ref,
             fc1w_ref, fc1b_ref, adjb_ref, fc2w_ref, fc2b_ref, edgb_ref,
             op_ref, of_ref, oa_ref):
        Wp, bp = [], []
        for c in range(6):
            w = fc1w_ref[c, 0] * adj_ref[0]
            b = fc1w_ref[c, 0] * adjb_ref[0]
            for k2 in range(1, 6):
                w = w + fc1w_ref[c, k2] * adj_ref[k2]
                b = b + fc1w_ref[c, k2] * adjb_ref[k2]
            Wp.append(w)
            bp.append(b + fc1b_ref[c])
        Wf, bf = [], []
        for c in range(5):
            w = fc2w_ref[c, 0] * edg_ref[0]
            b = fc2w_ref[c, 0] * edgb_ref[0]
            for k2 in range(1, 5):
                w = w + fc2w_ref[c, k2] * edg_ref[k2]
                b = b + fc2w_ref[c, k2] * edgb_ref[k2]
            Wf.append(w)
            bf.append(b + fc2b_ref[c])

        for bb in range(CG):
            btb = bt_ref[bb]
            es = []
            for c in range(6):
                t = lax.dot_general(btb, Wp[c], (((1,), (0,)), ((), ())),
                                    preferred_element_type=jnp.float32)
                es.append(_dotT(t, btb) + bp[c])
            mx = es[0]
            for c in range(1, 6):
                mx = jnp.maximum(mx, es[c])
            ssum = jnp.exp(es[0] - mx)
            for c in range(1, 6):
                ssum = ssum + jnp.exp(es[c] - mx)
            lse = mx + jnp.log(ssum)
            ep = jnp.stack([e - lse for e in es], axis=0)
            op_ref[bb] = jnp.transpose(ep, (1, 2, 0)).reshape(NA * NA, 6)
            fs = []
            for c in range(5):
                t = lax.dot_general(btb, Wf[c], (((1,), (0,)), ((), ())),
                                    preferred_element_type=jnp.float32)
                fs.append(_dotT(t, btb) + bf[c])
            of_ref[bb] = jnp.transpose(jnp.stack(fs, axis=0), (1, 2, 0))
            oa_ref[bb] = _dotT(btb, aw_ref[...]) + ab_ref[...]

    smem = pl.BlockSpec(memory_space=pltpu.SMEM)
    return pl.pallas_call(
        body,
        grid=(NCRY // CG,),
        in_specs=[
            pl.BlockSpec((CG, NA, D), lambda j: (j, 0, 0)),
            pl.BlockSpec((6, D, D), lambda j: (0, 0, 0)),
            pl.BlockSpec((5, D, D), lambda j: (0, 0, 0)),
            pl.BlockSpec((92, D), lambda j: (0, 0)),
            pl.BlockSpec((1, 92), lambda j: (0, 0)),
            smem, smem, smem, smem, smem, smem,
        ],
        out_specs=[
            pl.BlockSpec((CG, NA * NA, 6), lambda j: (j, 0, 0)),
            pl.BlockSpec((CG, NA, NA, 5), lambda j: (j, 0, 0, 0)),
            pl.BlockSpec((CG, NA, 92), lambda j: (j, 0, 0)),
        ],
        out_shape=[
            jax.ShapeDtypeStruct((NCRY, NA * NA, 6), jnp.float32),
            jax.ShapeDtypeStruct((NCRY, NA, NA, 5), jnp.float32),
            jax.ShapeDtypeStruct((NCRY, NA, 92), jnp.float32),
        ],
    )(bt3, fc_adj_W, fc_edge_W, fc_atom_W, fc_atom_b,
      fc1_W, fc1_b, fc_adj_b, fc2_W, fc2_b, fc_edge_b)


def kernel(atom_fea, nbr_fea, nbr_fea_idx, crystal_atom_idx, W_emb,
           fc_full_W, fc_full_b, bn1_g, bn1_b, bn2_g, bn2_b,
           fc_adj_W, fc_adj_b, fc1_W, fc1_b, fc_edge_W, fc_edge_b,
           fc2_W, fc2_b, fc_atom_W, fc_atom_b):
    idxT = jnp.pad(nbr_fea_idx.T.astype(jnp.int32), ((0, 0), (0, NPAD - N)))
    idx3 = idxT.reshape(NW, CPW, CHUNK)
    nbr2 = nbr_fea.reshape(N, M * DN)

    x = _embed(atom_fea, W_emb)
    for i in range(3):
        Ws = fc_full_W[i][:, :D]
        Wn = fc_full_W[i][:, D:2 * D]
        We = fc_full_W[i][:, 2 * D:]
        bv = _neighbor_transform(x, Wn)
        g0 = _sc_gather(bv, idx3).reshape(M, NPAD, 2 * D)
        ns, stats = _conv_layer(x, g0, nbr2, Ws, We,
                                fc_full_b[i][None, :],
                                bn1_g[i][None, :], bn1_b[i][None, :])
        x = _bn2_apply(x, ns, stats, bn2_g[i][None, :], bn2_b[i][None, :])

    bt3 = x.reshape(NCRY, NA, D)
    edge_p, edge_f, atom_out = _decoder(bt3, fc_adj_W, fc_edge_W, fc_atom_W,
                                        fc_atom_b[None, :], fc1_W, fc1_b,
                                        fc_adj_b, fc2_W, fc2_b, fc_edge_b)
    return edge_p, atom_out, edge_f


# speedup vs baseline: 1.8012x; 1.8012x over previous
import functools

import jax
import jax.numpy as jnp
from jax import lax
from jax.experimental import pallas as pl
from jax.experimental.pallas import tpu as pltpu
from jax.experimental.pallas import tpu_sc as plsc

N = 10000
M = 12
D = 64
DN = 41
NPAD = 10240
BPAD = M * NPAD
NW = 32
CPW = 30
CHUNK = 128
RB = 1000
NBC = N // RB
RB2 = 2000
CG = 4
NCRY = 200
NA = 50
EPS = 1e-5


def _softplus(v):
    return jnp.maximum(v, 0.0) + jnp.log(1.0 + jnp.exp(-jnp.abs(v)))


def _sigmoid(v):
    t = jnp.exp(-jnp.abs(v))
    return jnp.where(v >= 0, 1.0 / (1.0 + t), t / (1.0 + t))


def _dotT(a, b):
    return lax.dot_general(a, b, (((1,), (1,)), ((), ())),
                           preferred_element_type=jnp.float32)


def _sc_gather(bv, idx3):
    mesh = plsc.VectorSubcoreMesh(core_axis_name="c", subcore_axis_name="s")

    @functools.partial(
        pl.kernel,
        mesh=mesh,
        out_type=jax.ShapeDtypeStruct((BPAD, 2 * D), jnp.float32),
        scratch_types=[
            pltpu.VMEM((CPW, CHUNK), jnp.int32),
            pltpu.VMEM((CHUNK, 2 * D), jnp.float32),
            pltpu.VMEM((CHUNK, 2 * D), jnp.float32),
            pltpu.VMEM_SHARED((N, 2 * D), jnp.float32),
            pltpu.SemaphoreType.DMA,
            pltpu.SemaphoreType.DMA,
            pltpu.SemaphoreType.DMA,
            pltpu.SemaphoreType.DMA,
        ],
    )
    def k(x_hbm, idx_hbm, out_hbm, idx_v, buf0, buf1, tab, gi0, gi1, go0, go1):
        sid = lax.axis_index("s")
        wid = sid * 2 + lax.axis_index("c")
        base = wid * (CPW * CHUNK)

        @pl.when(sid < 10)
        def _():
            pltpu.sync_copy(x_hbm.at[pl.ds(sid * 1000, 1000)],
                            tab.at[pl.ds(sid * 1000, 1000)])

        pltpu.sync_copy(idx_hbm.at[wid], idx_v)
        plsc.subcore_barrier()

        def g_start(c, buf, sem):
            pltpu.async_copy(tab.at[idx_v.at[c]], buf, sem)

        def g_wait(c, buf, sem):
            pltpu.make_async_copy(tab.at[idx_v.at[c]], buf, sem).wait()

        def s_start(c, buf, sem):
            pltpu.async_copy(buf, out_hbm.at[pl.ds(base + c * CHUNK, CHUNK)], sem)

        def s_wait(buf, sem):
            pltpu.make_async_copy(buf, out_hbm.at[pl.ds(base, CHUNK)], sem).wait()

        g_start(0, buf0, gi0)

        def outer(o, carry):
            c0 = 2 * o
            @pl.when(o > 0)
            def _():
                s_wait(buf1, go1)

            g_start(c0 + 1, buf1, gi1)
            g_wait(c0, buf0, gi0)
            s_start(c0, buf0, go0)

            @pl.when(o < CPW // 2 - 1)
            def _():
                s_wait(buf0, go0)
                g_start(c0 + 2, buf0, gi0)

            g_wait(c0 + 1, buf1, gi1)
            s_start(c0 + 1, buf1, go1)
            return carry

        lax.fori_loop(0, CPW // 2, outer, 0)
        s_wait(buf0, go0)
        s_wait(buf1, go1)

    return k(bv, idx3)


def _embed(atom_fea, W_emb):
    def body(a_ref, w_ref, o_ref):
        o_ref[...] = _dotT(a_ref[...], w_ref[...])

    return pl.pallas_call(
        body,
        grid=(N // RB2,),
        in_specs=[
            pl.BlockSpec((RB2, 92), lambda j: (j, 0)),
            pl.BlockSpec((D, 92), lambda j: (0, 0)),
        ],
        out_specs=pl.BlockSpec((RB2, D), lambda j: (j, 0)),
        out_shape=jax.ShapeDtypeStruct((N, D), jnp.float32),
    )(atom_fea, W_emb)


def _neighbor_transform(x, Wn):
    def body(x_ref, w_ref, o_ref):
        o_ref[...] = _dotT(x_ref[...], w_ref[...])

    return pl.pallas_call(
        body,
        grid=(N // RB2,),
        in_specs=[
            pl.BlockSpec((RB2, D), lambda j: (j, 0)),
            pl.BlockSpec((2 * D, D), lambda j: (0, 0)),
        ],
        out_specs=pl.BlockSpec((RB2, 2 * D), lambda j: (j, 0)),
        out_shape=jax.ShapeDtypeStruct((N, 2 * D), jnp.float32),
    )(x, Wn)


def _conv_layer(x, g0, nbr2, Ws, We, bful, g1, b1):
    def body(x_ref, g0_ref, nf_ref, ws_ref, we_ref, b_ref,
             g1_ref, b1_ref, ns_ref, st_ref, acc):
        ph = pl.program_id(0)
        j = pl.program_id(1)

        @pl.when((ph == 0) & (j == 0))
        def _():
            acc[...] = jnp.zeros_like(acc)

        @pl.when((ph == 1) & (j == 0))
        def _():
            cnt = float(N * M)
            mean = acc[0:1, :] / cnt
            var = acc[1:2, :] / cnt - mean * mean
            acc[2:3, :] = mean
            acc[3:4, :] = lax.rsqrt(var + EPS)

        a = _dotT(x_ref[...], ws_ref[...]) + b_ref[...]

        nf = nf_ref[...]
        gs = []
        for m in range(M):
            g = a + g0_ref[m] + _dotT(nf[:, m * DN:(m + 1) * DN], we_ref[...])
            gs.append(g)

        @pl.when(ph == 0)
        def _():
            s = gs[0]
            s2 = gs[0] * gs[0]
            for m in range(1, M):
                s = s + gs[m]
                s2 = s2 + gs[m] * gs[m]
            acc[0:1, :] += jnp.sum(s, axis=0, keepdims=True)
            acc[1:2, :] += jnp.sum(s2, axis=0, keepdims=True)

        @pl.when(ph == 1)
        def _():
            mean = acc[2:3, :]
            scale = acc[3:4, :] * g1_ref[...]
            shift = b1_ref[...]
            ns = jnp.zeros((RB, D), jnp.float32)
            for m in range(M):
                ghat = (gs[m] - mean) * scale + shift
                ns = ns + _sigmoid(ghat[:, :D]) * _softplus(ghat[:, D:])
            ns_ref[...] = ns
            acc[4:5, :D] += jnp.sum(ns, axis=0, keepdims=True)
            acc[5:6, :D] += jnp.sum(ns * ns, axis=0, keepdims=True)

        @pl.when((ph == 1) & (j == NBC - 1))
        def _():
            st_ref[...] = acc[...]

    return pl.pallas_call(
        body,
        grid=(2, NBC),
        in_specs=[
            pl.BlockSpec((RB, D), lambda p, j: (j, 0)),
            pl.BlockSpec((M, RB, 2 * D), lambda p, j: (0, j, 0)),
            pl.BlockSpec((RB, M * DN), lambda p, j: (j, 0)),
            pl.BlockSpec((2 * D, D), lambda p, j: (0, 0)),
            pl.BlockSpec((2 * D, DN), lambda p, j: (0, 0)),
            pl.BlockSpec((1, 2 * D), lambda p, j: (0, 0)),
            pl.BlockSpec((1, 2 * D), lambda p, j: (0, 0)),
            pl.BlockSpec((1, 2 * D), lambda p, j: (0, 0)),
        ],
        out_specs=[
            pl.BlockSpec((RB, D), lambda p, j: (j, 0)),
            pl.BlockSpec((8, 2 * D), lambda p, j: (0, 0)),
        ],
        out_shape=[
            jax.ShapeDtypeStruct((N, D), jnp.float32),
            jax.ShapeDtypeStruct((8, 2 * D), jnp.float32),
        ],
        scratch_shapes=[pltpu.VMEM((8, 2 * D), jnp.float32)],
    )(x, g0, nbr2, Ws, We, bful, g1, b1)


def _bn2_apply(x, ns, stats, g2, b2):
    def body(x_ref, ns_ref, st_ref, g2_ref, b2_ref, o_ref):
        m2 = st_ref[4:5, :D] / float(N)
        v2 = st_ref[5:6, :D] / float(N) - m2 * m2
        sc = lax.rsqrt(v2 + EPS) * g2_ref[...]
        o_ref[...] = _softplus(x_ref[...] + (ns_ref[...] - m2) * sc + b2_ref[...])

    return pl.pallas_call(
        body,
        grid=(N // RB2,),
        in_specs=[
            pl.BlockSpec((RB2, D), lambda j: (j, 0)),
            pl.BlockSpec((RB2, D), lambda j: (j, 0)),
            pl.BlockSpec((8, 2 * D), lambda j: (0, 0)),
            pl.BlockSpec((1, D), lambda j: (0, 0)),
            pl.BlockSpec((1, D), lambda j: (0, 0)),
        ],
        out_specs=pl.BlockSpec((RB2, D), lambda j: (j, 0)),
        out_shape=jax.ShapeDtypeStruct((N, D), jnp.float32),
    )(x, ns, stats, g2, b2)


def _decoder(bt3, fc_adj_W, fc_edge_W, fc_atom_W, fc_atom_b,
             fc1_W, fc1_b, fc_adj_b, fc2_W, fc2_b, fc_edge_b):
    def body(bt_ref, adj_ref, edg_ref, aw_ref, ab_ref,
             fc1w_ref, fc1b_ref, adjb_ref, fc2w_ref, fc2b_ref, edgb_ref,
             op_ref, of_ref, oa_ref):
        Wp, bp = [], []
        for c in range(6):
            w = fc1w_ref[c, 0] * adj_ref[0]
            b = fc1w_ref[c, 0] * adjb_ref[0]
            for k2 in range(1, 6):
                w = w + fc1w_ref[c, k2] * adj_ref[k2]
                b = b + fc1w_ref[c, k2] * adjb_ref[k2]
            Wp.append(w)
            bp.append(b + fc1b_ref[c])
        Wf, bf = [], []
        for c in range(5):
            w = fc2w_ref[c, 0] * edg_ref[0]
            b = fc2w_ref[c, 0] * edgb_ref[0]
            for k2 in range(1, 5):
                w = w + fc2w_ref[c, k2] * edg_ref[k2]
                b = b + fc2w_ref[c, k2] * edgb_ref[k2]
            Wf.append(w)
            bf.append(b + fc2b_ref[c])

        for bb in range(CG):
            btb = bt_ref[bb]
            es = []
            for c in range(6):
                t = lax.dot_general(btb, Wp[c], (((1,), (0,)), ((), ())),
                                    preferred_element_type=jnp.float32)
                es.append(_dotT(t, btb) + bp[c])
            mx = es[0]
            for c in range(1, 6):
                mx = jnp.maximum(mx, es[c])
            ssum = jnp.exp(es[0] - mx)
            for c in range(1, 6):
                ssum = ssum + jnp.exp(es[c] - mx)
            lse = mx + jnp.log(ssum)
            for c in range(6):
                op_ref[c, bb] = es[c] - lse
            for c in range(5):
                t = lax.dot_general(btb, Wf[c], (((1,), (0,)), ((), ())),
                                    preferred_element_type=jnp.float32)
                of_ref[c, bb] = _dotT(t, btb) + bf[c]
            oa_ref[bb] = _dotT(btb, aw_ref[...]) + ab_ref[...]

    smem = pl.BlockSpec(memory_space=pltpu.SMEM)
    return pl.pallas_call(
        body,
        grid=(NCRY // CG,),
        in_specs=[
            pl.BlockSpec((CG, NA, D), lambda j: (j, 0, 0)),
            pl.BlockSpec((6, D, D), lambda j: (0, 0, 0)),
            pl.BlockSpec((5, D, D), lambda j: (0, 0, 0)),
            pl.BlockSpec((92, D), lambda j: (0, 0)),
            pl.BlockSpec((1, 92), lambda j: (0, 0)),
            smem, smem, smem, smem, smem, smem,
        ],
        out_specs=[
            pl.BlockSpec((6, CG, NA, NA), lambda j: (0, j, 0, 0)),
            pl.BlockSpec((5, CG, NA, NA), lambda j: (0, j, 0, 0)),
            pl.BlockSpec((CG, NA, 92), lambda j: (j, 0, 0)),
        ],
        out_shape=[
            jax.ShapeDtypeStruct((6, NCRY, NA, NA), jnp.float32),
            jax.ShapeDtypeStruct((5, NCRY, NA, NA), jnp.float32),
            jax.ShapeDtypeStruct((NCRY, NA, 92), jnp.float32),
        ],
    )(bt3, fc_adj_W, fc_edge_W, fc_atom_W, fc_atom_b,
      fc1_W, fc1_b, fc_adj_b, fc2_W, fc2_b, fc_edge_b)


def kernel(atom_fea, nbr_fea, nbr_fea_idx, crystal_atom_idx, W_emb,
           fc_full_W, fc_full_b, bn1_g, bn1_b, bn2_g, bn2_b,
           fc_adj_W, fc_adj_b, fc1_W, fc1_b, fc_edge_W, fc_edge_b,
           fc2_W, fc2_b, fc_atom_W, fc_atom_b):
    idxT = jnp.pad(nbr_fea_idx.T.astype(jnp.int32), ((0, 0), (0, NPAD - N)))
    idx3 = idxT.reshape(NW, CPW, CHUNK)
    nbr2 = nbr_fea.reshape(N, M * DN)

    x = _embed(atom_fea, W_emb)
    for i in range(3):
        Ws = fc_full_W[i][:, :D]
        Wn = fc_full_W[i][:, D:2 * D]
        We = fc_full_W[i][:, 2 * D:]
        bv = _neighbor_transform(x, Wn)
        g0 = _sc_gather(bv, idx3).reshape(M, NPAD, 2 * D)
        ns, stats = _conv_layer(x, g0, nbr2, Ws, We,
                                fc_full_b[i][None, :],
                                bn1_g[i][None, :], bn1_b[i][None, :])
        x = _bn2_apply(x, ns, stats, bn2_g[i][None, :], bn2_b[i][None, :])

    bt3 = x.reshape(NCRY, NA, D)
    op, of, oa = _decoder(bt3, fc_adj_W, fc_edge_W, fc_atom_W,
                          fc_atom_b[None, :], fc1_W, fc1_b, fc_adj_b,
                          fc2_W, fc2_b, fc_edge_b)
    edge_p = op.reshape(6, NCRY, NA * NA).transpose(1, 2, 0)
    edge_f = of.transpose(1, 2, 3, 0)
    atom_out = oa
    return edge_p, atom_out, edge_f


# speedup vs baseline: 1.8026x; 1.0007x over previous
import functools

import jax
import jax.numpy as jnp
from jax import lax
from jax.experimental import pallas as pl
from jax.experimental.pallas import tpu as pltpu
from jax.experimental.pallas import tpu_sc as plsc

N = 10000
M = 12
D = 64
DN = 41
NPAD = 10240
BPAD = M * NPAD
NW = 32
CPW = 30
CHUNK = 128
RB = 1000
NBC = N // RB
RB2 = 2000
CG = 4
NCRY = 200
NA = 50
EPS = 1e-5


def _softplus(v):
    return jnp.maximum(v, 0.0) + jnp.log(1.0 + jnp.exp(-jnp.abs(v)))


def _sigmoid(v):
    return 0.5 + 0.5 * jnp.tanh(0.5 * v)


def _dotT(a, b):
    return lax.dot_general(a, b, (((1,), (1,)), ((), ())),
                           preferred_element_type=jnp.float32)


def _sc_gather(bv, idx3):
    mesh = plsc.VectorSubcoreMesh(core_axis_name="c", subcore_axis_name="s")

    @functools.partial(
        pl.kernel,
        mesh=mesh,
        out_type=jax.ShapeDtypeStruct((BPAD, 2 * D), jnp.float32),
        scratch_types=[
            pltpu.VMEM((CPW, CHUNK), jnp.int32),
            pltpu.VMEM((CHUNK, 2 * D), jnp.float32),
            pltpu.VMEM((CHUNK, 2 * D), jnp.float32),
            pltpu.VMEM_SHARED((N, 2 * D), jnp.float32),
            pltpu.SemaphoreType.DMA,
            pltpu.SemaphoreType.DMA,
            pltpu.SemaphoreType.DMA,
            pltpu.SemaphoreType.DMA,
        ],
    )
    def k(x_hbm, idx_hbm, out_hbm, idx_v, buf0, buf1, tab, gi0, gi1, go0, go1):
        sid = lax.axis_index("s")
        wid = sid * 2 + lax.axis_index("c")
        base = wid * (CPW * CHUNK)

        @pl.when(sid < 10)
        def _():
            pltpu.sync_copy(x_hbm.at[pl.ds(sid * 1000, 1000)],
                            tab.at[pl.ds(sid * 1000, 1000)])

        pltpu.sync_copy(idx_hbm.at[wid], idx_v)
        plsc.subcore_barrier()

        def g_start(c, buf, sem):
            pltpu.async_copy(tab.at[idx_v.at[c]], buf, sem)

        def g_wait(c, buf, sem):
            pltpu.make_async_copy(tab.at[idx_v.at[c]], buf, sem).wait()

        def s_start(c, buf, sem):
            pltpu.async_copy(buf, out_hbm.at[pl.ds(base + c * CHUNK, CHUNK)], sem)

        def s_wait(buf, sem):
            pltpu.make_async_copy(buf, out_hbm.at[pl.ds(base, CHUNK)], sem).wait()

        g_start(0, buf0, gi0)

        def outer(o, carry):
            c0 = 2 * o
            @pl.when(o > 0)
            def _():
                s_wait(buf1, go1)

            g_start(c0 + 1, buf1, gi1)
            g_wait(c0, buf0, gi0)
            s_start(c0, buf0, go0)

            @pl.when(o < CPW // 2 - 1)
            def _():
                s_wait(buf0, go0)
                g_start(c0 + 2, buf0, gi0)

            g_wait(c0 + 1, buf1, gi1)
            s_start(c0 + 1, buf1, go1)
            return carry

        lax.fori_loop(0, CPW // 2, outer, 0)
        s_wait(buf0, go0)
        s_wait(buf1, go1)

    return k(bv, idx3)


def _embed(atom_fea, W_emb):
    def body(a_ref, w_ref, o_ref):
        o_ref[...] = _dotT(a_ref[...], w_ref[...])

    return pl.pallas_call(
        body,
        grid=(N // RB2,),
        in_specs=[
            pl.BlockSpec((RB2, 92), lambda j: (j, 0)),
            pl.BlockSpec((D, 92), lambda j: (0, 0)),
        ],
        out_specs=pl.BlockSpec((RB2, D), lambda j: (j, 0)),
        out_shape=jax.ShapeDtypeStruct((N, D), jnp.float32),
    )(atom_fea, W_emb)


def _neighbor_transform(x, Wn):
    def body(x_ref, w_ref, o_ref):
        o_ref[...] = _dotT(x_ref[...], w_ref[...])

    return pl.pallas_call(
        body,
        grid=(N // RB2,),
        in_specs=[
            pl.BlockSpec((RB2, D), lambda j: (j, 0)),
            pl.BlockSpec((2 * D, D), lambda j: (0, 0)),
        ],
        out_specs=pl.BlockSpec((RB2, 2 * D), lambda j: (j, 0)),
        out_shape=jax.ShapeDtypeStruct((N, 2 * D), jnp.float32),
    )(x, Wn)


def _conv_layer(x, g0, nbr2, Ws, We, bful, g1, b1):
    def body(x_ref, g0_ref, nf_ref, ws_ref, we_ref, b_ref,
             g1_ref, b1_ref, ns_ref, st_ref, acc):
        ph = pl.program_id(0)
        j = pl.program_id(1)

        @pl.when((ph == 0) & (j == 0))
        def _():
            acc[...] = jnp.zeros_like(acc)

        @pl.when((ph == 1) & (j == 0))
        def _():
            cnt = float(N * M)
            mean = acc[0:1, :] / cnt
            var = acc[1:2, :] / cnt - mean * mean
            acc[2:3, :] = mean
            acc[3:4, :] = lax.rsqrt(var + EPS)

        a = _dotT(x_ref[...], ws_ref[...]) + b_ref[...]

        def g_of(nf, m):
            return a + g0_ref[m] + _dotT(nf[:, m * DN:(m + 1) * DN], we_ref[...])

        @pl.when(ph == 0)
        def _():
            nf = nf_ref[...]
            g = g_of(nf, 0)
            s = g
            s2 = g * g
            for m in range(1, M):
                g = g_of(nf, m)
                s = s + g
                s2 = s2 + g * g
            acc[0:1, :] += jnp.sum(s, axis=0, keepdims=True)
            acc[1:2, :] += jnp.sum(s2, axis=0, keepdims=True)

        @pl.when(ph == 1)
        def _():
            nf = nf_ref[...]
            mean = acc[2:3, :]
            scale = acc[3:4, :] * g1_ref[...]
            shift = b1_ref[...]
            ns = jnp.zeros((RB, D), jnp.float32)
            for m in range(M):
                ghat = (g_of(nf, m) - mean) * scale + shift
                ns = ns + _sigmoid(ghat[:, :D]) * _softplus(ghat[:, D:])
            ns_ref[...] = ns
            acc[4:5, :D] += jnp.sum(ns, axis=0, keepdims=True)
            acc[5:6, :D] += jnp.sum(ns * ns, axis=0, keepdims=True)

        @pl.when((ph == 1) & (j == NBC - 1))
        def _():
            st_ref[...] = acc[...]

    return pl.pallas_call(
        body,
        grid=(2, NBC),
        in_specs=[
            pl.BlockSpec((RB, D), lambda p, j: (j, 0)),
            pl.BlockSpec((M, RB, 2 * D), lambda p, j: (0, j, 0)),
            pl.BlockSpec((RB, M * DN), lambda p, j: (j, 0)),
            pl.BlockSpec((2 * D, D), lambda p, j: (0, 0)),
            pl.BlockSpec((2 * D, DN), lambda p, j: (0, 0)),
            pl.BlockSpec((1, 2 * D), lambda p, j: (0, 0)),
            pl.BlockSpec((1, 2 * D), lambda p, j: (0, 0)),
            pl.BlockSpec((1, 2 * D), lambda p, j: (0, 0)),
        ],
        out_specs=[
            pl.BlockSpec((RB, D), lambda p, j: (j, 0)),
            pl.BlockSpec((8, 2 * D), lambda p, j: (0, 0)),
        ],
        out_shape=[
            jax.ShapeDtypeStruct((N, D), jnp.float32),
            jax.ShapeDtypeStruct((8, 2 * D), jnp.float32),
        ],
        scratch_shapes=[pltpu.VMEM((8, 2 * D), jnp.float32)],
    )(x, g0, nbr2, Ws, We, bful, g1, b1)


def _bn2_apply(x, ns, stats, g2, b2):
    def body(x_ref, ns_ref, st_ref, g2_ref, b2_ref, o_ref):
        m2 = st_ref[4:5, :D] / float(N)
        v2 = st_ref[5:6, :D] / float(N) - m2 * m2
        sc = lax.rsqrt(v2 + EPS) * g2_ref[...]
        o_ref[...] = _softplus(x_ref[...] + (ns_ref[...] - m2) * sc + b2_ref[...])

    return pl.pallas_call(
        body,
        grid=(N // RB2,),
        in_specs=[
            pl.BlockSpec((RB2, D), lambda j: (j, 0)),
            pl.BlockSpec((RB2, D), lambda j: (j, 0)),
            pl.BlockSpec((8, 2 * D), lambda j: (0, 0)),
            pl.BlockSpec((1, D), lambda j: (0, 0)),
            pl.BlockSpec((1, D), lambda j: (0, 0)),
        ],
        out_specs=pl.BlockSpec((RB2, D), lambda j: (j, 0)),
        out_shape=jax.ShapeDtypeStruct((N, D), jnp.float32),
    )(x, ns, stats, g2, b2)


def _decoder(bt3, fc_adj_W, fc_edge_W, fc_atom_W, fc_atom_b,
             fc1_W, fc1_b, fc_adj_b, fc2_W, fc2_b, fc_edge_b):
    def body(bt_ref, adj_ref, edg_ref, aw_ref, ab_ref,
             fc1w_ref, fc1b_ref, adjb_ref, fc2w_ref, fc2b_ref, edgb_ref,
             op_ref, of_ref, oa_ref):
        Wp, bp = [], []
        for c in range(6):
            w = fc1w_ref[c, 0] * adj_ref[0]
            b = fc1w_ref[c, 0] * adjb_ref[0]
            for k2 in range(1, 6):
                w = w + fc1w_ref[c, k2] * adj_ref[k2]
                b = b + fc1w_ref[c, k2] * adjb_ref[k2]
            Wp.append(w)
            bp.append(b + fc1b_ref[c])
        Wf, bf = [], []
        for c in range(5):
            w = fc2w_ref[c, 0] * edg_ref[0]
            b = fc2w_ref[c, 0] * edgb_ref[0]
            for k2 in range(1, 5):
                w = w + fc2w_ref[c, k2] * edg_ref[k2]
                b = b + fc2w_ref[c, k2] * edgb_ref[k2]
            Wf.append(w)
            bf.append(b + fc2b_ref[c])

        for bb in range(CG):
            btb = bt_ref[bb]
            es = []
            for c in range(6):
                t = lax.dot_general(btb, Wp[c], (((1,), (0,)), ((), ())),
                                    preferred_element_type=jnp.float32)
                es.append(_dotT(t, btb) + bp[c])
            mx = es[0]
            for c in range(1, 6):
                mx = jnp.maximum(mx, es[c])
            ssum = jnp.exp(es[0] - mx)
            for c in range(1, 6):
                ssum = ssum + jnp.exp(es[c] - mx)
            lse = mx + jnp.log(ssum)
            for c in range(6):
                op_ref[c, bb] = es[c] - lse
            for c in range(5):
                t = lax.dot_general(btb, Wf[c], (((1,), (0,)), ((), ())),
                                    preferred_element_type=jnp.float32)
                of_ref[c, bb] = _dotT(t, btb) + bf[c]
            oa_ref[bb] = _dotT(btb, aw_ref[...]) + ab_ref[...]

    smem = pl.BlockSpec(memory_space=pltpu.SMEM)
    return pl.pallas_call(
        body,
        grid=(NCRY // CG,),
        in_specs=[
            pl.BlockSpec((CG, NA, D), lambda j: (j, 0, 0)),
            pl.BlockSpec((6, D, D), lambda j: (0, 0, 0)),
            pl.BlockSpec((5, D, D), lambda j: (0, 0, 0)),
            pl.BlockSpec((92, D), lambda j: (0, 0)),
            pl.BlockSpec((1, 92), lambda j: (0, 0)),
            smem, smem, smem, smem, smem, smem,
        ],
        out_specs=[
            pl.BlockSpec((6, CG, NA, NA), lambda j: (0, j, 0, 0)),
            pl.BlockSpec((5, CG, NA, NA), lambda j: (0, j, 0, 0)),
            pl.BlockSpec((CG, NA, 92), lambda j: (j, 0, 0)),
        ],
        out_shape=[
            jax.ShapeDtypeStruct((6, NCRY, NA, NA), jnp.float32),
            jax.ShapeDtypeStruct((5, NCRY, NA, NA), jnp.float32),
            jax.ShapeDtypeStruct((NCRY, NA, 92), jnp.float32),
        ],
    )(bt3, fc_adj_W, fc_edge_W, fc_atom_W, fc_atom_b,
      fc1_W, fc1_b, fc_adj_b, fc2_W, fc2_b, fc_edge_b)


def kernel(atom_fea, nbr_fea, nbr_fea_idx, crystal_atom_idx, W_emb,
           fc_full_W, fc_full_b, bn1_g, bn1_b, bn2_g, bn2_b,
           fc_adj_W, fc_adj_b, fc1_W, fc1_b, fc_edge_W, fc_edge_b,
           fc2_W, fc2_b, fc_atom_W, fc_atom_b):
    idxT = jnp.pad(nbr_fea_idx.T.astype(jnp.int32), ((0, 0), (0, NPAD - N)))
    idx3 = idxT.reshape(NW, CPW, CHUNK)
    nbr2 = nbr_fea.reshape(N, M * DN)

    x = _embed(atom_fea, W_emb)
    for i in range(3):
        Ws = fc_full_W[i][:, :D]
        Wn = fc_full_W[i][:, D:2 * D]
        We = fc_full_W[i][:, 2 * D:]
        bv = _neighbor_transform(x, Wn)
        g0 = _sc_gather(bv, idx3).reshape(M, NPAD, 2 * D)
        ns, stats = _conv_layer(x, g0, nbr2, Ws, We,
                                fc_full_b[i][None, :],
                                bn1_g[i][None, :], bn1_b[i][None, :])
        x = _bn2_apply(x, ns, stats, bn2_g[i][None, :], bn2_b[i][None, :])

    bt3 = x.reshape(NCRY, NA, D)
    op, of, oa = _decoder(bt3, fc_adj_W, fc_edge_W, fc_atom_W,
                          fc_atom_b[None, :], fc1_W, fc1_b, fc_adj_b,
                          fc2_W, fc2_b, fc_edge_b)
    edge_p = op.reshape(6, NCRY, NA * NA).transpose(1, 2, 0)
    edge_f = of.transpose(1, 2, 3, 0)
    atom_out = oa
    return edge_p, atom_out, edge_f
